# baseline (speedup 1.0000x reference)
import functools

import jax
import jax.numpy as jnp
from jax import lax
from jax.experimental import pallas as pl
from jax.experimental.pallas import tpu as pltpu
from jax.experimental.pallas import tpu_sc as plsc

NS = 16
CH = 128
V = 100001
D = 32
RB = 512
NBLK = 49
V4 = NBLK * RB


def _pack_body(t0_ref, t1_ref, t2_ref, t3_ref, out_ref):
    out_ref[...] = jnp.concatenate(
        [t0_ref[...], t1_ref[...], t2_ref[...], t3_ref[...]], axis=1)


def _pack_table(table):
    specs = [
        pl.BlockSpec((RB, D), lambda i, q=q: (q * NBLK + i, 0))
        for q in range(4)
    ]
    return pl.pallas_call(
        _pack_body,
        grid=(NBLK,),
        in_specs=specs,
        out_specs=pl.BlockSpec((RB, 4 * D), lambda i: (i, 0)),
        out_shape=jax.ShapeDtypeStruct((V4, 4 * D), jnp.float32),
    )(table, table, table, table)


def _sc_gather(umod, dmod, upacked, opacked):
    B = umod.shape[0]
    b_per_tile = B // NS
    n_ch = b_per_tile // CH

    uid3 = umod.reshape(NS, n_ch, CH)
    did3 = dmod.reshape(NS, n_ch, CH)

    mesh = plsc.VectorSubcoreMesh(core_axis_name="c", subcore_axis_name="s")

    @functools.partial(
        pl.kernel,
        out_type=(jax.ShapeDtypeStruct((B, CH), jnp.float32),
                  jax.ShapeDtypeStruct((B, CH), jnp.float32)),
        mesh=mesh,
        scratch_types=[
            pltpu.VMEM((n_ch, CH), jnp.int32),
            pltpu.VMEM((CH, CH), jnp.float32),
            pltpu.VMEM((CH, CH), jnp.float32),
            pltpu.SemaphoreType.DMA,
            pltpu.SemaphoreType.DMA,
        ],
        compiler_params=pltpu.CompilerParams(use_tc_tiling_on_sc=True),
    )
    def gather_kernel(uid_hbm, did_hbm, upk_hbm, opk_hbm,
                      uout_hbm, oout_hbm,
                      idxv, gbuf0, gbuf1, sem0, sem1):
        c = lax.axis_index("c")
        s = lax.axis_index("s")

        def work(idx_hbm, pk_ref, out_ref):
            pltpu.sync_copy(idx_hbm.at[s], idxv)
            base = s * b_per_tile
            gbufs = (gbuf0, gbuf1)
            sems = (sem0, sem1)
            copies = [None] * n_ch
            for j in range(n_ch):
                copies[j] = pltpu.async_copy(
                    pk_ref.at[idxv.at[j]], gbufs[j % 2], sems[j % 2])
                if j >= 1:
                    copies[j - 1].wait()
                    pltpu.sync_copy(
                        gbufs[(j - 1) % 2],
                        out_ref.at[pl.ds(base + (j - 1) * CH, CH)])
            copies[n_ch - 1].wait()
            pltpu.sync_copy(
                gbufs[(n_ch - 1) % 2],
                out_ref.at[pl.ds(base + (n_ch - 1) * CH, CH)])

        @pl.when(c == 0)
        def _():
            work(uid_hbm, upk_hbm, uout_hbm)

        @pl.when(c == 1)
        def _():
            work(did_hbm, opk_hbm, oout_hbm)

    return gather_kernel(uid3, did3, upacked, opacked)


def _mlp_body(us_ref, os_ref, uq_ref, oq_ref, w1u_ref, w1o_ref, b1_ref,
              w2_ref, b2_ref, w3t_ref, b3_ref, out_ref):
    uq = uq_ref[...]
    oq = oq_ref[...]
    u = jnp.zeros((us_ref.shape[0], D), jnp.float32)
    o = jnp.zeros((os_ref.shape[0], D), jnp.float32)
    for q in range(4):
        u = jnp.where(uq == q, us_ref[:, q * D:(q + 1) * D], u)
        o = jnp.where(oq == q, os_ref[:, q * D:(q + 1) * D], o)
    h1 = jnp.dot(u, w1u_ref[...], preferred_element_type=jnp.float32)
    h1 = h1 + jnp.dot(o, w1o_ref[...], preferred_element_type=jnp.float32)
    h1 = jnp.maximum(h1 + b1_ref[...], 0.0)
    h2 = jnp.dot(h1, w2_ref[...], preferred_element_type=jnp.float32)
    h2 = jnp.maximum(h2 + b2_ref[...], 0.0)
    out_ref[...] = (jnp.sum(h2 * w3t_ref[...], axis=1, keepdims=True)
                    + b3_ref[...])


def _mlp(u_sup, o_sup, uq, oq, W1, b1, W2, b2, W3, b3, chunk=2048):
    B = u_sup.shape[0]
    H1 = W1.shape[1]
    H2 = W2.shape[1]
    w1u = W1[:D]
    w1o = W1[D:]
    b1r = b1.reshape(1, H1)
    b2r = b2.reshape(1, H2)
    w3t = W3.reshape(1, H2)
    b3r = b3.reshape(1, 1)
    grid = (B // chunk,)
    return pl.pallas_call(
        _mlp_body,
        grid=grid,
        in_specs=[
            pl.BlockSpec((chunk, CH), lambda i: (i, 0)),
            pl.BlockSpec((chunk, CH), lambda i: (i, 0)),
            pl.BlockSpec((chunk, 1), lambda i: (i, 0)),
            pl.BlockSpec((chunk, 1), lambda i: (i, 0)),
            pl.BlockSpec((D, H1), lambda i: (0, 0)),
            pl.BlockSpec((D, H1), lambda i: (0, 0)),
            pl.BlockSpec((1, H1), lambda i: (0, 0)),
            pl.BlockSpec((H1, H2), lambda i: (0, 0)),
            pl.BlockSpec((1, H2), lambda i: (0, 0)),
            pl.BlockSpec((1, H2), lambda i: (0, 0)),
            pl.BlockSpec((1, 1), lambda i: (0, 0)),
        ],
        out_specs=pl.BlockSpec((chunk, 1), lambda i: (i, 0)),
        out_shape=jax.ShapeDtypeStruct((B, 1), jnp.float32),
    )(u_sup, o_sup, uq, oq, w1u, w1o, b1r, W2, b2r, w3t, b3r)


def kernel(user_id, destination, user_table, origin_table,
           W1, b1, W2, b2, W3, b3):
    uid = user_id.astype(jnp.int32)
    did = destination.astype(jnp.int32)
    umod = uid % V4
    dmod = did % V4
    uq = (uid // V4).reshape(-1, 1)
    oq = (did // V4).reshape(-1, 1)
    upacked = _pack_table(user_table)
    opacked = _pack_table(origin_table)
    u_sup, o_sup = _sc_gather(umod, dmod, upacked, opacked)
    return _mlp(u_sup, o_sup, uq, oq, W1, b1, W2, b2, W3, b3)


# speedup vs baseline: 1.6076x; 1.6076x over previous
import functools

import jax
import jax.numpy as jnp
from jax import lax
from jax.experimental import pallas as pl
from jax.experimental.pallas import tpu as pltpu
from jax.experimental.pallas import tpu_sc as plsc

NC, NS = 2, 16
NW = NC * NS
CH = 128


def _sc_gather_one(idx, table):
    B = idx.shape[0]
    D = table.shape[1]
    b_per_w = B // NW
    n_ch = b_per_w // CH

    idx3 = idx.reshape(NW, n_ch, CH).astype(jnp.int32)

    mesh = plsc.VectorSubcoreMesh(core_axis_name="c", subcore_axis_name="s")

    @functools.partial(
        pl.kernel,
        out_type=jax.ShapeDtypeStruct((B, D), jnp.float32),
        mesh=mesh,
        scratch_types=[
            pltpu.VMEM((n_ch, CH), jnp.int32),
            pltpu.VMEM((b_per_w, D), jnp.float32),
            pltpu.SemaphoreType.DMA,
        ],
        compiler_params=pltpu.CompilerParams(use_tc_tiling_on_sc=False),
    )
    def gather_kernel(idx_hbm, tab_hbm, out_hbm, idx_v, rows_v, sem):
        wid = lax.axis_index("s") * NC + lax.axis_index("c")
        base = wid * b_per_w
        pltpu.sync_copy(idx_hbm.at[wid], idx_v)
        copies = []
        for j in range(n_ch):
            copies.append(pltpu.async_copy(
                tab_hbm.at[idx_v.at[j]], rows_v.at[pl.ds(j * CH, CH)], sem))
        for c in copies:
            c.wait()
        pltpu.sync_copy(rows_v, out_hbm.at[pl.ds(base, b_per_w)])

    return gather_kernel(idx3, table)


def _mlp_body(u_ref, o_ref, w1u_ref, w1o_ref, b1_ref, w2_ref, b2_ref,
              w3t_ref, b3_ref, out_ref):
    h1 = jnp.dot(u_ref[...], w1u_ref[...], preferred_element_type=jnp.float32)
    h1 = h1 + jnp.dot(o_ref[...], w1o_ref[...],
                      preferred_element_type=jnp.float32)
    h1 = jnp.maximum(h1 + b1_ref[...], 0.0)
    h2 = jnp.dot(h1, w2_ref[...], preferred_element_type=jnp.float32)
    h2 = jnp.maximum(h2 + b2_ref[...], 0.0)
    out_ref[...] = (jnp.sum(h2 * w3t_ref[...], axis=1, keepdims=True)
                    + b3_ref[...])


def _mlp(u_emb, o_emb, W1, b1, W2, b2, W3, b3, chunk=2048):
    B, D = u_emb.shape
    H1 = W1.shape[1]
    H2 = W2.shape[1]
    w1u = W1[:D]
    w1o = W1[D:]
    b1r = b1.reshape(1, H1)
    b2r = b2.reshape(1, H2)
    w3t = W3.reshape(1, H2)
    b3r = b3.reshape(1, 1)
    grid = (B // chunk,)
    return pl.pallas_call(
        _mlp_body,
        grid=grid,
        in_specs=[
            pl.BlockSpec((chunk, D), lambda i: (i, 0)),
            pl.BlockSpec((chunk, D), lambda i: (i, 0)),
            pl.BlockSpec((D, H1), lambda i: (0, 0)),
            pl.BlockSpec((D, H1), lambda i: (0, 0)),
            pl.BlockSpec((1, H1), lambda i: (0, 0)),
            pl.BlockSpec((H1, H2), lambda i: (0, 0)),
            pl.BlockSpec((1, H2), lambda i: (0, 0)),
            pl.BlockSpec((1, H2), lambda i: (0, 0)),
            pl.BlockSpec((1, 1), lambda i: (0, 0)),
        ],
        out_specs=pl.BlockSpec((chunk, 1), lambda i: (i, 0)),
        out_shape=jax.ShapeDtypeStruct((B, 1), jnp.float32),
    )(u_emb, o_emb, w1u, w1o, b1r, W2, b2r, w3t, b3r)


def kernel(user_id, destination, user_table, origin_table,
           W1, b1, W2, b2, W3, b3):
    u_emb = _sc_gather_one(user_id, user_table)
    o_emb = _sc_gather_one(destination, origin_table)
    return _mlp(u_emb, o_emb, W1, b1, W2, b2, W3, b3)
